# baseline (speedup 1.0000x reference)
import functools
import math

import jax
import jax.numpy as jnp
from jax import lax
from jax.experimental import pallas as pl
from jax.experimental.pallas import tpu as pltpu
from jax.experimental.pallas import tpu_sc as plsc

D_MODEL = 64
SCALE = math.sqrt(D_MODEL)
NUM_CORES = 2
NUM_SUBCORES = 16
NUM_WORKERS = NUM_CORES * NUM_SUBCORES
CHUNK = 512


@functools.partial(jax.jit, static_argnums=(2, 3))
def _embed(tokens_flat, table, b_per_w, n_chunks):
    mesh = plsc.VectorSubcoreMesh(core_axis_name="c", subcore_axis_name="s")
    B = b_per_w * NUM_WORKERS

    @functools.partial(
        pl.kernel,
        out_type=jax.ShapeDtypeStruct((B, D_MODEL), jnp.float32),
        mesh=mesh,
        scratch_types=[
            pltpu.VMEM((CHUNK,), jnp.int32),
            pltpu.VMEM((CHUNK, D_MODEL), jnp.float32),
            pltpu.SemaphoreType.DMA,
        ],
        compiler_params=pltpu.CompilerParams(use_tc_tiling_on_sc=False),
    )
    def body(tok_hbm, table_hbm, out_hbm, idx_v, rows_v, sem):
        wid = lax.axis_index("s") * NUM_CORES + lax.axis_index("c")
        base = wid * b_per_w

        def chunk_body(g, carry):
            off = base + g * CHUNK
            pltpu.sync_copy(tok_hbm.at[pl.ds(off, CHUNK)], idx_v)
            pltpu.async_copy(table_hbm.at[idx_v], rows_v, sem).wait()

            def mul_body(i, c):
                for j in range(D_MODEL // 16):
                    sl = pl.ds(j * 16, 16)
                    rows_v[i, sl] = rows_v[i, sl] * SCALE
                return c

            lax.fori_loop(0, CHUNK, mul_body, 0, unroll=4)
            pltpu.sync_copy(rows_v, out_hbm.at[pl.ds(off, CHUNK)])
            return carry

        lax.fori_loop(0, n_chunks, chunk_body, 0)

    return body(tokens_flat, table)


def kernel(tokens, table):
    B = tokens.shape[0] * tokens.shape[1]
    tok = tokens.reshape(B).astype(jnp.int32)
    b_per_w = B // NUM_WORKERS
    n_chunks = b_per_w // CHUNK
    out = _embed(tok, table, b_per_w, n_chunks)
    return out.reshape(tokens.shape[0], tokens.shape[1], D_MODEL)


# speedup vs baseline: 1.0892x; 1.0892x over previous
import functools
import math

import jax
import jax.numpy as jnp
from jax import lax
from jax.experimental import pallas as pl
from jax.experimental.pallas import tpu as pltpu
from jax.experimental.pallas import tpu_sc as plsc

D_MODEL = 64
SCALE = math.sqrt(D_MODEL)
NUM_CORES = 2
NUM_SUBCORES = 16
NUM_WORKERS = NUM_CORES * NUM_SUBCORES
CHUNK = 256
NBUF = 5
AHEAD = NBUF - 2


@functools.partial(jax.jit, static_argnums=(2, 3))
def _embed(tokens_flat, table, b_per_w, n_chunks):
    mesh = plsc.VectorSubcoreMesh(core_axis_name="c", subcore_axis_name="s")
    B = b_per_w * NUM_WORKERS

    @functools.partial(
        pl.kernel,
        out_type=jax.ShapeDtypeStruct((B, D_MODEL), jnp.float32),
        mesh=mesh,
        scratch_types=[
            pltpu.VMEM((NBUF, CHUNK), jnp.int32),
            pltpu.VMEM((NBUF, CHUNK, D_MODEL), jnp.float32),
        ]
        + [pltpu.SemaphoreType.DMA] * (2 * NBUF),
        compiler_params=pltpu.CompilerParams(use_tc_tiling_on_sc=False),
    )
    def body(tok_hbm, table_hbm, out_hbm, idx_v, rows_v, *sems):
        gsem = sems[:NBUF]
        ssem = sems[NBUF:]
        wid = lax.axis_index("s") * NUM_CORES + lax.axis_index("c")
        base = wid * b_per_w

        def issue_gather(g, slot):
            off = base + g * CHUNK
            pltpu.sync_copy(tok_hbm.at[pl.ds(off, CHUNK)], idx_v.at[slot])
            pltpu.async_copy(
                table_hbm.at[idx_v.at[slot]], rows_v.at[slot], gsem[slot]
            )

        for g in range(AHEAD):
            issue_gather(g, g % NBUF)

        def outer(t, carry):
            for j in range(NBUF):
                g = t * NBUF + j
                pltpu.make_async_copy(
                    table_hbm.at[idx_v.at[j]], rows_v.at[j], gsem[j]
                ).wait()

                def mul_body(i, c):
                    for q in range(D_MODEL // 16):
                        sl = pl.ds(q * 16, 16)
                        rows_v[j, i, sl] = rows_v[j, i, sl] * SCALE
                    return c

                lax.fori_loop(0, CHUNK, mul_body, 0, unroll=4)

                off = base + g * CHUNK
                pltpu.async_copy(
                    rows_v.at[j], out_hbm.at[pl.ds(off, CHUNK)], ssem[j]
                )

                nxt = g + AHEAD
                c_slot = (j + AHEAD) % NBUF

                @pl.when(nxt < n_chunks)
                def _():
                    @pl.when(nxt >= NBUF)
                    def _():
                        pltpu.make_async_copy(
                            rows_v.at[c_slot],
                            out_hbm.at[pl.ds(base, CHUNK)],
                            ssem[c_slot],
                        ).wait()

                    issue_gather(nxt, c_slot)

            return carry

        lax.fori_loop(0, n_chunks // NBUF, outer, 0)

        for j in range(NBUF):
            pltpu.make_async_copy(
                rows_v.at[j], out_hbm.at[pl.ds(base, CHUNK)], ssem[j]
            ).wait()

    return body(tokens_flat, table)


def kernel(tokens, table):
    B = tokens.shape[0] * tokens.shape[1]
    tok = tokens.reshape(B).astype(jnp.int32)
    b_per_w = B // NUM_WORKERS
    n_chunks = b_per_w // CHUNK
    out = _embed(tok, table, b_per_w, n_chunks)
    return out.reshape(tokens.shape[0], tokens.shape[1], D_MODEL)
